# baseline (speedup 1.0000x reference)
import functools

import jax
import jax.numpy as jnp
from jax import lax
from jax.experimental import pallas as pl
from jax.experimental.pallas import tpu as pltpu
from jax.experimental.pallas import tpu_sc as plsc

VOCAB = 1_000_000
RANK = 32
EMB = 64
L_SEQ = 50
B_SZ = 4096

NC, NS = 2, 16
NW = NC * NS
B_CHK = B_SZ // NW

T_COLS = 65536
T_SUB = T_COLS // 4
T_GRID = -(-VOCAB // T_COLS)


def _tc_repack_table(a_t):
    eye = jnp.eye(4 * RANK, dtype=jnp.float32)

    def body(a_ref, e_ref, o_ref):
        acc = None
        for q in range(4):
            sub = a_ref[:, T_SUB * q : T_SUB * (q + 1)]
            z = lax.dot_general(
                sub,
                e_ref[RANK * q : RANK * (q + 1), :],
                (((0,), (0,)), ((), ())),
                preferred_element_type=jnp.float32,
            )
            acc = z if acc is None else acc + z
        o_ref[...] = acc

    return pl.pallas_call(
        body,
        grid=(T_GRID,),
        in_specs=[
            pl.BlockSpec((RANK, T_COLS), lambda g: (0, g)),
            pl.BlockSpec((4 * RANK, 4 * RANK), lambda g: (0, 0)),
        ],
        out_specs=pl.BlockSpec((T_SUB, 4 * RANK), lambda g: (g, 0)),
        out_shape=jax.ShapeDtypeStruct((T_GRID * T_SUB, 4 * RANK), jnp.float32),
    )(a_t, eye)


def _sc_gather_lb(table, idx_sc):
    mesh = plsc.VectorSubcoreMesh(
        core_axis_name="c", subcore_axis_name="s", num_cores=NC, num_subcores=NS
    )

    @functools.partial(
        pl.kernel,
        out_type=jax.ShapeDtypeStruct((L_SEQ * B_SZ // 4, 128), jnp.float32),
        mesh=mesh,
        scratch_types=[
            pltpu.VMEM((L_SEQ, B_CHK), jnp.int32),
            pltpu.VMEM((B_CHK, RANK), jnp.float32),
            pltpu.VMEM((B_CHK, RANK), jnp.float32),
            pltpu.VMEM((B_CHK // 4, 128), jnp.float32),
            pltpu.VMEM((B_CHK // 4, 128), jnp.float32),
            pltpu.SemaphoreType.DMA,
            pltpu.SemaphoreType.DMA,
            pltpu.SemaphoreType.DMA,
            pltpu.SemaphoreType.DMA,
        ],
        compiler_params=pltpu.CompilerParams(use_tc_tiling_on_sc=False),
    )
    def k(table_hbm, idx_hbm, out_hbm, xs, g0, g1, s0, s1, sg0, sg1, so0, so1):
        wid = lax.axis_index("s") * NC + lax.axis_index("c")
        pltpu.sync_copy(idx_hbm.at[wid], xs)

        def fire_gather(l, g, sg):
            pltpu.async_copy(table_hbm.at[xs.at[l]], g, sg)

        def repack(g, s):
            for r in range(B_CHK // 4):
                for j in range(8):
                    s[r, pl.ds(16 * j, 16)] = g[
                        4 * r + j // 2, pl.ds(16 * (j % 2), 16)
                    ]

        def out_dst(l):
            return out_hbm.at[pl.ds(B_SZ // 4 * l + RANK * wid, RANK)]

        fire_gather(0, g0, sg0)
        fire_gather(1, g1, sg1)

        def body(i, carry):
            l0 = 2 * i
            l1 = 2 * i + 1

            pltpu.make_async_copy(table_hbm.at[xs.at[l0]], g0, sg0).wait()

            @pl.when(i > 0)
            def _():
                pltpu.make_async_copy(s0, out_dst(l0), so0).wait()

            repack(g0, s0)
            pltpu.async_copy(s0, out_dst(l0), so0)

            @pl.when(i < L_SEQ // 2 - 1)
            def _():
                fire_gather(l0 + 2, g0, sg0)

            pltpu.make_async_copy(table_hbm.at[xs.at[l1]], g1, sg1).wait()

            @pl.when(i > 0)
            def _():
                pltpu.make_async_copy(s1, out_dst(l1), so1).wait()

            repack(g1, s1)
            pltpu.async_copy(s1, out_dst(l1), so1)

            @pl.when(i < L_SEQ // 2 - 1)
            def _():
                fire_gather(l1 + 2, g1, sg1)

            return carry

        lax.fori_loop(0, L_SEQ // 2, body, 0)
        pltpu.make_async_copy(s0, out_dst(L_SEQ - 2), so0).wait()
        pltpu.make_async_copy(s1, out_dst(L_SEQ - 1), so1).wait()

    return k(table, idx_sc)


def _tc_project_t(glb4, bt):
    def mm(b_ref, g_ref, o_ref):
        for q in range(4):
            gq = g_ref[:, RANK * q : RANK * (q + 1)]
            o_ref[:, 1024 * q : 1024 * (q + 1)] = lax.dot_general(
                b_ref[...],
                gq,
                (((1,), (1,)), ((), ())),
                preferred_element_type=jnp.float32,
            )

    return pl.pallas_call(
        mm,
        grid=(L_SEQ,),
        in_specs=[
            pl.BlockSpec((EMB, RANK), lambda i: (0, 0)),
            pl.BlockSpec((B_SZ // 4, 128), lambda i: (i, 0)),
        ],
        out_specs=pl.BlockSpec((EMB, B_SZ), lambda i: (i, 0)),
        out_shape=jax.ShapeDtypeStruct((L_SEQ * EMB, B_SZ), jnp.float32),
    )(bt, glb4)


def kernel(x, A, B):
    i = x.astype(jnp.int32)
    g = i // T_COLS
    rem = i - g * T_COLS
    q = rem // T_SUB
    r = rem - q * T_SUB
    xj = 4 * (T_SUB * g + r) + q

    idx_sc = (
        xj.reshape(4, NW, B_CHK // 4, L_SEQ)
        .transpose(1, 3, 2, 0)
        .reshape(NW, L_SEQ, B_CHK)
    )

    table = _tc_repack_table(A.T).reshape(T_GRID * T_SUB * 4, RANK)
    glb4 = _sc_gather_lb(table, idx_sc)
    ot = _tc_project_t(glb4, B.T)
    return ot.reshape(L_SEQ, EMB, B_SZ).transpose(2, 0, 1)


# speedup vs baseline: 1.0159x; 1.0159x over previous
import functools

import jax
import jax.numpy as jnp
from jax import lax
from jax.experimental import pallas as pl
from jax.experimental.pallas import tpu as pltpu
from jax.experimental.pallas import tpu_sc as plsc

VOCAB = 1_000_000
RANK = 32
EMB = 64
L_SEQ = 50
B_SZ = 4096

NC, NS = 2, 16
NW = NC * NS
B_CHK = B_SZ // NW

T_COLS = 32768
T_SUB = T_COLS // 4
T_GRID = -(-VOCAB // T_COLS)


def _tc_repack_table(a_t):
    eye = jnp.eye(4 * RANK, dtype=jnp.float32)

    def body(a_ref, e_ref, o_ref):
        acc = None
        for q in range(4):
            sub = a_ref[:, T_SUB * q : T_SUB * (q + 1)]
            z = lax.dot_general(
                sub,
                e_ref[RANK * q : RANK * (q + 1), :],
                (((0,), (0,)), ((), ())),
                preferred_element_type=jnp.float32,
            )
            acc = z if acc is None else acc + z
        o_ref[...] = acc

    return pl.pallas_call(
        body,
        grid=(T_GRID,),
        in_specs=[
            pl.BlockSpec((RANK, T_COLS), lambda g: (0, g)),
            pl.BlockSpec((4 * RANK, 4 * RANK), lambda g: (0, 0)),
        ],
        out_specs=pl.BlockSpec((T_SUB, 4 * RANK), lambda g: (g, 0)),
        out_shape=jax.ShapeDtypeStruct((T_GRID * T_SUB, 4 * RANK), jnp.float32),
    )(a_t, eye)


def _sc_gather_lb(table, idx_sc):
    mesh = plsc.VectorSubcoreMesh(
        core_axis_name="c", subcore_axis_name="s", num_cores=NC, num_subcores=NS
    )

    @functools.partial(
        pl.kernel,
        out_type=jax.ShapeDtypeStruct((L_SEQ * B_SZ // 4, 128), jnp.float32),
        mesh=mesh,
        scratch_types=[
            pltpu.VMEM((L_SEQ, B_CHK), jnp.int32),
            pltpu.VMEM((B_CHK, RANK), jnp.float32),
            pltpu.VMEM((B_CHK, RANK), jnp.float32),
            pltpu.VMEM((B_CHK // 4, 128), jnp.float32),
            pltpu.VMEM((B_CHK // 4, 128), jnp.float32),
            pltpu.SemaphoreType.DMA,
            pltpu.SemaphoreType.DMA,
            pltpu.SemaphoreType.DMA,
            pltpu.SemaphoreType.DMA,
        ],
        compiler_params=pltpu.CompilerParams(use_tc_tiling_on_sc=False),
    )
    def k(table_hbm, idx_hbm, out_hbm, xs, g0, g1, s0, s1, sg0, sg1, so0, so1):
        wid = lax.axis_index("s") * NC + lax.axis_index("c")
        pltpu.sync_copy(idx_hbm.at[wid], xs)

        def fire_gather(l, g, sg):
            pltpu.async_copy(table_hbm.at[xs.at[l]], g, sg)

        def repack(g, s):
            for r in range(B_CHK // 4):
                for j in range(8):
                    s[r, pl.ds(16 * j, 16)] = g[
                        4 * r + j // 2, pl.ds(16 * (j % 2), 16)
                    ]

        def out_dst(l):
            return out_hbm.at[pl.ds(B_SZ // 4 * l + RANK * wid, RANK)]

        fire_gather(0, g0, sg0)
        fire_gather(1, g1, sg1)

        def body(i, carry):
            l0 = 2 * i
            l1 = 2 * i + 1

            pltpu.make_async_copy(table_hbm.at[xs.at[l0]], g0, sg0).wait()

            @pl.when(i > 0)
            def _():
                pltpu.make_async_copy(s0, out_dst(l0), so0).wait()

            repack(g0, s0)
            pltpu.async_copy(s0, out_dst(l0), so0)

            @pl.when(i < L_SEQ // 2 - 1)
            def _():
                fire_gather(l0 + 2, g0, sg0)

            pltpu.make_async_copy(table_hbm.at[xs.at[l1]], g1, sg1).wait()

            @pl.when(i > 0)
            def _():
                pltpu.make_async_copy(s1, out_dst(l1), so1).wait()

            repack(g1, s1)
            pltpu.async_copy(s1, out_dst(l1), so1)

            @pl.when(i < L_SEQ // 2 - 1)
            def _():
                fire_gather(l1 + 2, g1, sg1)

            return carry

        lax.fori_loop(0, L_SEQ // 2, body, 0)
        pltpu.make_async_copy(s0, out_dst(L_SEQ - 2), so0).wait()
        pltpu.make_async_copy(s1, out_dst(L_SEQ - 1), so1).wait()

    return k(table, idx_sc)


def _tc_project_t(glb4, bt):
    def mm(b_ref, g_ref, o_ref):
        for q in range(4):
            gq = g_ref[:, RANK * q : RANK * (q + 1)]
            o_ref[:, 1024 * q : 1024 * (q + 1)] = lax.dot_general(
                b_ref[...],
                gq,
                (((1,), (1,)), ((), ())),
                preferred_element_type=jnp.float32,
            )

    return pl.pallas_call(
        mm,
        grid=(L_SEQ,),
        in_specs=[
            pl.BlockSpec((EMB, RANK), lambda i: (0, 0)),
            pl.BlockSpec((B_SZ // 4, 128), lambda i: (i, 0)),
        ],
        out_specs=pl.BlockSpec((EMB, B_SZ), lambda i: (i, 0)),
        out_shape=jax.ShapeDtypeStruct((L_SEQ * EMB, B_SZ), jnp.float32),
    )(bt, glb4)


def kernel(x, A, B):
    i = x.astype(jnp.int32)
    g = i // T_COLS
    rem = i - g * T_COLS
    q = rem // T_SUB
    r = rem - q * T_SUB
    xj = 4 * (T_SUB * g + r) + q

    idx_sc = (
        xj.reshape(4, NW, B_CHK // 4, L_SEQ)
        .transpose(1, 3, 2, 0)
        .reshape(NW, L_SEQ, B_CHK)
    )

    table = _tc_repack_table(A.T).reshape(T_GRID * T_SUB * 4, RANK)
    glb4 = _sc_gather_lb(table, idx_sc)
    ot = _tc_project_t(glb4, B.T)
    return ot.reshape(L_SEQ, EMB, B_SZ).transpose(2, 0, 1)


# speedup vs baseline: 1.0706x; 1.0539x over previous
import functools

import jax
import jax.numpy as jnp
from jax import lax
from jax.experimental import pallas as pl
from jax.experimental.pallas import tpu as pltpu
from jax.experimental.pallas import tpu_sc as plsc

VOCAB = 1_000_000
RANK = 32
EMB = 64
L_SEQ = 50
B_SZ = 4096

NC, NS = 2, 16
NW = NC * NS
B_CHK = B_SZ // NW

T_COLS = 32768
T_SUB = T_COLS // 4
T_GRID = -(-VOCAB // T_COLS)


def _tc_repack_table(a_t):
    eye = jnp.eye(4 * RANK, dtype=jnp.float32)

    def body(a_ref, e_ref, o_ref):
        acc = None
        for q in range(4):
            sub = a_ref[:, T_SUB * q : T_SUB * (q + 1)]
            z = lax.dot_general(
                sub,
                e_ref[RANK * q : RANK * (q + 1), :],
                (((0,), (0,)), ((), ())),
                preferred_element_type=jnp.float32,
            )
            acc = z if acc is None else acc + z
        o_ref[...] = acc

    return pl.pallas_call(
        body,
        grid=(T_GRID,),
        in_specs=[
            pl.BlockSpec((RANK, T_COLS), lambda g: (0, g)),
            pl.BlockSpec((4 * RANK, 4 * RANK), lambda g: (0, 0)),
        ],
        out_specs=pl.BlockSpec((T_SUB, 4 * RANK), lambda g: (g, 0)),
        out_shape=jax.ShapeDtypeStruct((T_GRID * T_SUB, 4 * RANK), jnp.float32),
    )(a_t, eye)


def _sc_gather_lb(table, idx_sc):
    mesh = plsc.VectorSubcoreMesh(
        core_axis_name="c", subcore_axis_name="s", num_cores=NC, num_subcores=NS
    )

    @functools.partial(
        pl.kernel,
        out_type=jax.ShapeDtypeStruct((L_SEQ * B_SZ // 4, 128), jnp.float32),
        mesh=mesh,
        scratch_types=[
            pltpu.VMEM((L_SEQ, B_CHK), jnp.int32),
            pltpu.VMEM((B_CHK, RANK), jnp.float32),
            pltpu.VMEM((B_CHK, RANK), jnp.float32),
            pltpu.VMEM((B_CHK // 4, 128), jnp.float32),
            pltpu.VMEM((B_CHK // 4, 128), jnp.float32),
            pltpu.SemaphoreType.DMA,
            pltpu.SemaphoreType.DMA,
            pltpu.SemaphoreType.DMA,
            pltpu.SemaphoreType.DMA,
        ],
        compiler_params=pltpu.CompilerParams(use_tc_tiling_on_sc=False),
    )
    def k(table_hbm, idx_hbm, out_hbm, xs, g0, g1, s0, s1, sg0, sg1, so0, so1):
        wid = lax.axis_index("s") * NC + lax.axis_index("c")
        pltpu.sync_copy(idx_hbm.at[wid], xs)

        def fire_gather(l, g, sg):
            pltpu.async_copy(table_hbm.at[xs.at[l]], g, sg)

        def repack(g, s):
            for r in range(B_CHK // 4):
                for j in range(8):
                    s[r, pl.ds(16 * j, 16)] = g[
                        4 * r + j // 2, pl.ds(16 * (j % 2), 16)
                    ]

        def out_dst(l):
            return out_hbm.at[pl.ds(B_SZ // 4 * l + RANK * wid, RANK)]

        fire_gather(0, g0, sg0)
        fire_gather(1, g1, sg1)

        def body(i, carry):
            l0 = 2 * i
            l1 = 2 * i + 1

            pltpu.make_async_copy(table_hbm.at[xs.at[l0]], g0, sg0).wait()

            @pl.when(i > 0)
            def _():
                pltpu.make_async_copy(s0, out_dst(l0), so0).wait()

            repack(g0, s0)
            pltpu.async_copy(s0, out_dst(l0), so0)

            @pl.when(i < L_SEQ // 2 - 1)
            def _():
                fire_gather(l0 + 2, g0, sg0)

            pltpu.make_async_copy(table_hbm.at[xs.at[l1]], g1, sg1).wait()

            @pl.when(i > 0)
            def _():
                pltpu.make_async_copy(s1, out_dst(l1), so1).wait()

            repack(g1, s1)
            pltpu.async_copy(s1, out_dst(l1), so1)

            @pl.when(i < L_SEQ // 2 - 1)
            def _():
                fire_gather(l1 + 2, g1, sg1)

            return carry

        lax.fori_loop(0, L_SEQ // 2, body, 0)
        pltpu.make_async_copy(s0, out_dst(L_SEQ - 2), so0).wait()
        pltpu.make_async_copy(s1, out_dst(L_SEQ - 1), so1).wait()

    return k(table, idx_sc)


def _tc_project_t(glb4, bt):
    def mm(b_ref, g_ref, o_ref):
        for h in range(2):
            for q in range(4):
                gq = g_ref[
                    1024 * h : 1024 * (h + 1), RANK * q : RANK * (q + 1)
                ]
                o_ref[
                    EMB * h : EMB * (h + 1), 1024 * q : 1024 * (q + 1)
                ] = lax.dot_general(
                    b_ref[...],
                    gq,
                    (((1,), (1,)), ((), ())),
                    preferred_element_type=jnp.float32,
                )

    return pl.pallas_call(
        mm,
        grid=(L_SEQ // 2,),
        in_specs=[
            pl.BlockSpec((EMB, RANK), lambda i: (0, 0)),
            pl.BlockSpec((B_SZ // 2, 128), lambda i: (i, 0)),
        ],
        out_specs=pl.BlockSpec((2 * EMB, B_SZ), lambda i: (i, 0)),
        out_shape=jax.ShapeDtypeStruct((L_SEQ * EMB, B_SZ), jnp.float32),
    )(bt, glb4)


def kernel(x, A, B):
    i = x.astype(jnp.int32)
    g = i // T_COLS
    rem = i - g * T_COLS
    q = rem // T_SUB
    r = rem - q * T_SUB
    xj = 4 * (T_SUB * g + r) + q

    idx_sc = (
        xj.reshape(4, NW, B_CHK // 4, L_SEQ)
        .transpose(1, 3, 2, 0)
        .reshape(NW, L_SEQ, B_CHK)
    )

    table = _tc_repack_table(A.T).reshape(T_GRID * T_SUB * 4, RANK)
    glb4 = _sc_gather_lb(table, idx_sc)
    ot = _tc_project_t(glb4, B.T)
    return ot.reshape(L_SEQ, EMB, B_SZ).transpose(2, 0, 1)
